# Initial kernel scaffold; baseline (speedup 1.0000x reference)
import jax
import jax.numpy as jnp
from jax.experimental import pallas as pl


def kernel(x, x_net, edge_index_node_net, edge_index_net_node, num_instances, batch, enc_W1, enc_b1, enc_W2, enc_b2, encn_W1, encn_b1, encn_W2, encn_b2, conv_W, conv_b, conv_root, reconv_W, reconv_b, reconv_root, ln_gamma, ln_beta):
    raise NotImplementedError("write your pallas kernel here")


# speedup vs baseline: 10.4564x; 10.4564x over previous
import functools

import jax
import jax.numpy as jnp
from jax import lax
from jax.experimental import pallas as pl
from jax.experimental.pallas import tpu as pltpu
from jax.experimental.pallas import tpu_sc as plsc

N_INST = 8000
N_NET = 2000
N_TOT = N_INST + N_NET
E = 320000
EMB = 128
NUM_LAYER = 2

NW = 32
EPW = E // NW
EB = 80
NBLK = EPW // EB
RPW = 624

_sc_mesh = plsc.VectorSubcoreMesh(core_axis_name="c", subcore_axis_name="s")
_sc_params = pltpu.CompilerParams(needs_layout_passes=False)


def _leaky(x):
    return jnp.where(x >= 0, x, 0.1 * x)


def _mlp_body(x_ref, w1_ref, b1_ref, w2_ref, b2_ref, o_ref):
    h = jnp.dot(x_ref[...], w1_ref[...], preferred_element_type=jnp.float32)
    h = _leaky(h + b1_ref[...])
    h = jnp.dot(h, w2_ref[...], preferred_element_type=jnp.float32)
    o_ref[...] = _leaky(h + b2_ref[...])


def _mlp(x, w1t, b1, w2t, b2, blk):
    n, kin = x.shape
    h1 = w1t.shape[1]
    return pl.pallas_call(
        _mlp_body,
        grid=(n // blk,),
        in_specs=[
            pl.BlockSpec((blk, kin), lambda i: (i, 0)),
            pl.BlockSpec((kin, h1), lambda i: (0, 0)),
            pl.BlockSpec((1, h1), lambda i: (0, 0)),
            pl.BlockSpec((h1, EMB), lambda i: (0, 0)),
            pl.BlockSpec((1, EMB), lambda i: (0, 0)),
        ],
        out_specs=pl.BlockSpec((blk, EMB), lambda i: (i, 0)),
        out_shape=jax.ShapeDtypeStruct((n, EMB), jnp.float32),
    )(x, w1t, b1, w2t, b2)


def _deg_body(p_ref, o_ref):
    p = p_ref[...]
    deg1 = jnp.sum(p[:NW], axis=0) + 1.0
    deg2 = jnp.sum(p[NW:], axis=0) + 1.0
    o_ref[0, :] = lax.rsqrt(deg1)
    o_ref[1, :] = lax.rsqrt(deg2)


def _deg_reduce(partials):
    return pl.pallas_call(
        _deg_body,
        out_shape=jax.ShapeDtypeStruct((2, N_TOT), jnp.float32),
    )(partials)


def _pre_body(h_ref, wt_ref, b_ref, root_ref, dinv_ref, u_ref, z_ref):
    xl = jnp.dot(h_ref[...], wt_ref[...], preferred_element_type=jnp.float32)
    xl = xl + b_ref[...]
    dv = dinv_ref[...]
    u_ref[...] = dv * jnp.maximum(xl, 0.0)
    z_ref[...] = (dv * dv) * jnp.maximum(xl + root_ref[...], 0.0)


def _pre(h, wt, b, root, dinv, blk=1000):
    return pl.pallas_call(
        _pre_body,
        grid=(N_TOT // blk,),
        in_specs=[
            pl.BlockSpec((blk, EMB), lambda i: (i, 0)),
            pl.BlockSpec((EMB, EMB), lambda i: (0, 0)),
            pl.BlockSpec((1, EMB), lambda i: (0, 0)),
            pl.BlockSpec((1, EMB), lambda i: (0, 0)),
            pl.BlockSpec((blk, 1), lambda i: (i, 0)),
        ],
        out_specs=[
            pl.BlockSpec((blk, EMB), lambda i: (i, 0)),
            pl.BlockSpec((blk, EMB), lambda i: (i, 0)),
        ],
        out_shape=[
            jax.ShapeDtypeStruct((N_TOT, EMB), jnp.float32),
            jax.ShapeDtypeStruct((N_TOT, EMB), jnp.float32),
        ],
    )(h, wt, b, root, dinv)


def _post_ln_body(acc_ref, z_ref, dinv_ref, g_ref, b_ref, o_ref):
    cval = dinv_ref[...] * (acc_ref[0] + acc_ref[1]) + z_ref[...]
    mu = jnp.mean(cval, axis=-1, keepdims=True)
    d = cval - mu
    var = jnp.mean(d * d, axis=-1, keepdims=True)
    y = d * lax.rsqrt(var + 1e-5) * g_ref[...] + b_ref[...]
    o_ref[...] = jnp.maximum(y, 0.0)


def _post_ln(acc, z, dinv, gamma, beta, blk=1000):
    return pl.pallas_call(
        _post_ln_body,
        grid=(N_TOT // blk,),
        in_specs=[
            pl.BlockSpec((2, blk, EMB), lambda i: (0, i, 0)),
            pl.BlockSpec((blk, EMB), lambda i: (i, 0)),
            pl.BlockSpec((blk, 1), lambda i: (i, 0)),
            pl.BlockSpec((1, EMB), lambda i: (0, 0)),
            pl.BlockSpec((1, EMB), lambda i: (0, 0)),
        ],
        out_specs=pl.BlockSpec((blk, EMB), lambda i: (i, 0)),
        out_shape=jax.ShapeDtypeStruct((N_TOT, EMB), jnp.float32),
    )(acc, z, dinv, gamma, beta)


def _post_res_body(acc_ref, z_ref, dinv_ref, hprev_ref, o_ref):
    cval = dinv_ref[...] * (acc_ref[0] + acc_ref[1]) + z_ref[...]
    o_ref[...] = cval + hprev_ref[...]


def _post_res(acc, z, dinv, hprev, blk=1000):
    return pl.pallas_call(
        _post_res_body,
        grid=(N_TOT // blk,),
        in_specs=[
            pl.BlockSpec((2, blk, EMB), lambda i: (0, i, 0)),
            pl.BlockSpec((blk, EMB), lambda i: (i, 0)),
            pl.BlockSpec((blk, 1), lambda i: (i, 0)),
            pl.BlockSpec((blk, EMB), lambda i: (i, 0)),
        ],
        out_specs=pl.BlockSpec((blk, EMB), lambda i: (i, 0)),
        out_shape=jax.ShapeDtypeStruct((N_TOT, EMB), jnp.float32),
    )(acc, z, dinv, hprev)


@functools.partial(
    pl.kernel,
    out_type=jax.ShapeDtypeStruct((2 * NW, N_TOT), jnp.float32),
    mesh=_sc_mesh,
    compiler_params=_sc_params,
    scratch_types=[
        pltpu.VMEM((N_TOT,), jnp.float32),
        pltpu.VMEM((EPW,), jnp.int32),
    ],
)
def _hist_sc(row1_hbm, row2_hbm, out_hbm, hist_v, idx_v):
    c = lax.axis_index("c")
    s = lax.axis_index("s")
    wid = c * 16 + s
    base = wid * EPW
    ones = jnp.ones((16,), jnp.float32)
    zeros = jnp.zeros((16,), jnp.float32)
    for which, row_hbm in enumerate((row1_hbm, row2_hbm)):
        def zbody(i, _):
            hist_v[pl.ds(i * 16, 16)] = zeros
            return 0
        lax.fori_loop(0, N_TOT // 16, zbody, 0)
        pltpu.sync_copy(row_hbm.at[pl.ds(base, EPW)], idx_v)

        def body(i, _):
            idx = idx_v[pl.ds(i * 16, 16)]
            plsc.addupdate_scatter(hist_v, [idx], ones)
            return 0
        lax.fori_loop(0, EPW // 16, body, 0)
        pltpu.sync_copy(hist_v, out_hbm.at[which * NW + wid])


@functools.partial(
    pl.kernel,
    out_type=jax.ShapeDtypeStruct((2, N_TOT, EMB), jnp.float32),
    mesh=_sc_mesh,
    compiler_params=_sc_params,
    scratch_types=[
        pltpu.VMEM_SHARED((N_TOT, EMB), jnp.float32),
        pltpu.VMEM((EB,), jnp.int32),
        pltpu.VMEM((EB,), jnp.int32),
        pltpu.VMEM((EB, EMB), jnp.float32),
        pltpu.SemaphoreType.DMA,
    ],
)
def _seg_sc(u_hbm, row_hbm, col_hbm, zeros_hbm, out_hbm,
            acc_sh, row_v, col_v, g_v, sem):
    c = lax.axis_index("c")
    s = lax.axis_index("s")
    pltpu.sync_copy(zeros_hbm.at[pl.ds(0, RPW)], acc_sh.at[pl.ds(s * RPW, RPW)])

    @pl.when(s == 15)
    def _():
        pltpu.sync_copy(zeros_hbm.at[pl.ds(0, 16)],
                        acc_sh.at[pl.ds(16 * RPW, 16)])
    plsc.subcore_barrier()
    base = (c * 16 + s) * EPW

    def body(j, _):
        off = base + j * EB
        pltpu.sync_copy(row_hbm.at[pl.ds(off, EB)], row_v)
        pltpu.sync_copy(col_hbm.at[pl.ds(off, EB)], col_v)
        pltpu.async_copy(u_hbm.at[row_v], g_v, sem).wait()
        pltpu.sync_copy(g_v, acc_sh.at[col_v], add=True)
        return 0
    lax.fori_loop(0, NBLK, body, 0)
    plsc.subcore_barrier()
    pltpu.sync_copy(acc_sh.at[pl.ds(s * RPW, RPW)],
                    out_hbm.at[c, pl.ds(s * RPW, RPW)])

    @pl.when(s == 15)
    def _():
        pltpu.sync_copy(acc_sh.at[pl.ds(16 * RPW, 16)],
                        out_hbm.at[c, pl.ds(16 * RPW, 16)])


def kernel(x, x_net, edge_index_node_net, edge_index_net_node, num_instances,
           batch, enc_W1, enc_b1, enc_W2, enc_b2, encn_W1, encn_b1, encn_W2,
           encn_b2, conv_W, conv_b, conv_root, reconv_W, reconv_b, reconv_root,
           ln_gamma, ln_beta):
    f32 = jnp.float32
    row1, col1 = edge_index_node_net[0], edge_index_node_net[1]
    row2, col2 = edge_index_net_node[0], edge_index_net_node[1]

    xp = jnp.pad(x, ((0, 0), (0, 128 - x.shape[1])))
    xnp = jnp.pad(x_net, ((0, 0), (0, 128 - x_net.shape[1])))
    w1t = jnp.pad(enc_W1.T, ((0, 128 - enc_W1.shape[1]), (0, 0)))
    wn1t = jnp.pad(encn_W1.T, ((0, 128 - encn_W1.shape[1]), (0, 0)))
    x_inst = _mlp(xp, w1t, enc_b1[None, :], enc_W2.T, enc_b2[None, :], 1000)
    xn = _mlp(xnp, wn1t, encn_b1[None, :], encn_W2.T, encn_b2[None, :], 1000)
    h0 = jnp.concatenate([x_inst, xn], axis=0)

    partials = _hist_sc(row1, row2)
    dinv_all = _deg_reduce(partials)
    dinv1 = dinv_all[0][:, None]
    dinv2 = dinv_all[1][:, None]

    zeros_slab = jnp.zeros((RPW, EMB), f32)

    h_list = [h0]
    h = h0
    for l in range(NUM_LAYER):
        u1, z1 = _pre(h, conv_W[l].T, conv_b[l][None, :],
                      conv_root[l][None, :], dinv1)
        acc1 = _seg_sc(u1, row1, col1, zeros_slab)
        h_mid = _post_ln(acc1, z1, dinv1, ln_gamma[l][None, :],
                         ln_beta[l][None, :])
        u2, z2 = _pre(h_mid, reconv_W[l].T, reconv_b[l][None, :],
                      reconv_root[l][None, :], dinv2)
        acc2 = _seg_sc(u2, row2, col2, zeros_slab)
        h = _post_res(acc2, z2, dinv2, h)
        h_list.append(h)

    return jnp.concatenate(h_list, axis=1)


# speedup vs baseline: 22.9987x; 2.1995x over previous
import functools

import jax
import jax.numpy as jnp
from jax import lax
from jax.experimental import pallas as pl
from jax.experimental.pallas import tpu as pltpu
from jax.experimental.pallas import tpu_sc as plsc

N_INST = 8000
N_NET = 2000
N_TOT = N_INST + N_NET
E = 320000
EMB = 128
NUM_LAYER = 2

NW = 32
EPW = E // NW
EB = 100
NBLK = EPW // EB
RPW = 624

_sc_mesh = plsc.VectorSubcoreMesh(core_axis_name="c", subcore_axis_name="s")
_sc_params = pltpu.CompilerParams(needs_layout_passes=False)
_sc_flat_params = pltpu.CompilerParams(needs_layout_passes=False,
                                       use_tc_tiling_on_sc=False)


def _leaky(x):
    return jnp.where(x >= 0, x, 0.1 * x)


def _mlp_body(x_ref, w1_ref, b1_ref, w2_ref, b2_ref, o_ref):
    h = jnp.dot(x_ref[...], w1_ref[...], preferred_element_type=jnp.float32)
    h = _leaky(h + b1_ref[...])
    h = jnp.dot(h, w2_ref[...], preferred_element_type=jnp.float32)
    o_ref[...] = _leaky(h + b2_ref[...])


def _mlp(x, w1t, b1, w2t, b2, blk):
    n, kin = x.shape
    h1 = w1t.shape[1]
    return pl.pallas_call(
        _mlp_body,
        grid=(n // blk,),
        in_specs=[
            pl.BlockSpec((blk, kin), lambda i: (i, 0)),
            pl.BlockSpec((kin, h1), lambda i: (0, 0)),
            pl.BlockSpec((1, h1), lambda i: (0, 0)),
            pl.BlockSpec((h1, EMB), lambda i: (0, 0)),
            pl.BlockSpec((1, EMB), lambda i: (0, 0)),
        ],
        out_specs=pl.BlockSpec((blk, EMB), lambda i: (i, 0)),
        out_shape=jax.ShapeDtypeStruct((n, EMB), jnp.float32),
    )(x, w1t, b1, w2t, b2)


def _deg_body(p_ref, o_ref):
    p = p_ref[...]
    deg1 = jnp.sum(p[:NW], axis=0) + 1.0
    deg2 = jnp.sum(p[NW:], axis=0) + 1.0
    o_ref[0, :] = lax.rsqrt(deg1)
    o_ref[1, :] = lax.rsqrt(deg2)


def _deg_reduce(partials):
    return pl.pallas_call(
        _deg_body,
        out_shape=jax.ShapeDtypeStruct((2, N_TOT), jnp.float32),
    )(partials)


def _pre_body(h_ref, wt_ref, b_ref, root_ref, dinv_ref, u_ref, z_ref):
    xl = jnp.dot(h_ref[...], wt_ref[...], preferred_element_type=jnp.float32)
    xl = xl + b_ref[...]
    dv = dinv_ref[...]
    u_ref[...] = dv * jnp.maximum(xl, 0.0)
    z_ref[...] = (dv * dv) * jnp.maximum(xl + root_ref[...], 0.0)


def _pre(h, wt, b, root, dinv, blk=1000):
    return pl.pallas_call(
        _pre_body,
        grid=(N_TOT // blk,),
        in_specs=[
            pl.BlockSpec((blk, EMB), lambda i: (i, 0)),
            pl.BlockSpec((EMB, EMB), lambda i: (0, 0)),
            pl.BlockSpec((1, EMB), lambda i: (0, 0)),
            pl.BlockSpec((1, EMB), lambda i: (0, 0)),
            pl.BlockSpec((blk, 1), lambda i: (i, 0)),
        ],
        out_specs=[
            pl.BlockSpec((blk, EMB), lambda i: (i, 0)),
            pl.BlockSpec((blk, EMB), lambda i: (i, 0)),
        ],
        out_shape=[
            jax.ShapeDtypeStruct((N_TOT, EMB), jnp.float32),
            jax.ShapeDtypeStruct((N_TOT, EMB), jnp.float32),
        ],
    )(h, wt, b, root, dinv)


def _post_ln_body(acc_ref, z_ref, dinv_ref, g_ref, b_ref, o_ref):
    cval = dinv_ref[...] * (acc_ref[0] + acc_ref[1]) + z_ref[...]
    mu = jnp.mean(cval, axis=-1, keepdims=True)
    d = cval - mu
    var = jnp.mean(d * d, axis=-1, keepdims=True)
    y = d * lax.rsqrt(var + 1e-5) * g_ref[...] + b_ref[...]
    o_ref[...] = jnp.maximum(y, 0.0)


def _post_ln(acc, z, dinv, gamma, beta, blk=1000):
    return pl.pallas_call(
        _post_ln_body,
        grid=(N_TOT // blk,),
        in_specs=[
            pl.BlockSpec((2, blk, EMB), lambda i: (0, i, 0)),
            pl.BlockSpec((blk, EMB), lambda i: (i, 0)),
            pl.BlockSpec((blk, 1), lambda i: (i, 0)),
            pl.BlockSpec((1, EMB), lambda i: (0, 0)),
            pl.BlockSpec((1, EMB), lambda i: (0, 0)),
        ],
        out_specs=pl.BlockSpec((blk, EMB), lambda i: (i, 0)),
        out_shape=jax.ShapeDtypeStruct((N_TOT, EMB), jnp.float32),
    )(acc, z, dinv, gamma, beta)


def _post_res_body(acc_ref, z_ref, dinv_ref, hprev_ref, o_ref):
    cval = dinv_ref[...] * (acc_ref[0] + acc_ref[1]) + z_ref[...]
    o_ref[...] = cval + hprev_ref[...]


def _post_res(acc, z, dinv, hprev, blk=1000):
    return pl.pallas_call(
        _post_res_body,
        grid=(N_TOT // blk,),
        in_specs=[
            pl.BlockSpec((2, blk, EMB), lambda i: (0, i, 0)),
            pl.BlockSpec((blk, EMB), lambda i: (i, 0)),
            pl.BlockSpec((blk, 1), lambda i: (i, 0)),
            pl.BlockSpec((blk, EMB), lambda i: (i, 0)),
        ],
        out_specs=pl.BlockSpec((blk, EMB), lambda i: (i, 0)),
        out_shape=jax.ShapeDtypeStruct((N_TOT, EMB), jnp.float32),
    )(acc, z, dinv, hprev)


@functools.partial(
    pl.kernel,
    out_type=jax.ShapeDtypeStruct((2 * NW, N_TOT), jnp.float32),
    mesh=_sc_mesh,
    compiler_params=_sc_params,
    scratch_types=[
        pltpu.VMEM((N_TOT,), jnp.float32),
        pltpu.VMEM((EPW,), jnp.int32),
    ],
)
def _hist_sc(row1_hbm, row2_hbm, out_hbm, hist_v, idx_v):
    c = lax.axis_index("c")
    s = lax.axis_index("s")
    wid = c * 16 + s
    base = wid * EPW
    ones = jnp.ones((16,), jnp.float32)
    zeros = jnp.zeros((16,), jnp.float32)
    for which, row_hbm in enumerate((row1_hbm, row2_hbm)):
        def zbody(i, _):
            hist_v[pl.ds(i * 16, 16)] = zeros
            return 0
        lax.fori_loop(0, N_TOT // 16, zbody, 0)
        pltpu.sync_copy(row_hbm.at[pl.ds(base, EPW)], idx_v)

        def body(i, _):
            idx = idx_v[pl.ds(i * 16, 16)]
            plsc.addupdate_scatter(hist_v, [idx], ones)
            return 0
        lax.fori_loop(0, EPW // 16, body, 0)
        pltpu.sync_copy(hist_v, out_hbm.at[which * NW + wid])


NBUF = 2


@functools.partial(
    pl.kernel,
    out_type=jax.ShapeDtypeStruct((2, N_TOT, EMB), jnp.float32),
    mesh=_sc_mesh,
    compiler_params=_sc_flat_params,
    scratch_types=[
        pltpu.VMEM_SHARED((N_TOT, EMB), jnp.float32),
        pltpu.VMEM((NBLK, EB), jnp.int32),
        pltpu.VMEM((NBLK, EB), jnp.int32),
        pltpu.VMEM((NBUF, EB, EMB), jnp.float32),
        pltpu.SemaphoreType.DMA((NBUF,)),
    ],
)
def _seg_sc(u_hbm, row_hbm, col_hbm, zeros_hbm, out_hbm,
            acc_sh, row_v, col_v, g_v, gsem):
    c = lax.axis_index("c")
    s = lax.axis_index("s")
    pltpu.sync_copy(zeros_hbm.at[pl.ds(0, RPW)], acc_sh.at[pl.ds(s * RPW, RPW)])

    @pl.when(s == 15)
    def _():
        pltpu.sync_copy(zeros_hbm.at[pl.ds(0, 16)],
                        acc_sh.at[pl.ds(16 * RPW, 16)])

    wid = c * 16 + s
    pltpu.sync_copy(row_hbm.at[wid], row_v)
    pltpu.sync_copy(col_hbm.at[wid], col_v)
    plsc.subcore_barrier()

    for k in range(NBUF):
        pltpu.async_copy(u_hbm.at[row_v.at[k]], g_v.at[k], gsem.at[k])

    def body(t, _):
        for k in range(NBUF):
            j = t * NBUF + k
            pltpu.make_async_copy(u_hbm.at[row_v.at[j]], g_v.at[k],
                                  gsem.at[k]).wait()
            pltpu.sync_copy(g_v.at[k], acc_sh.at[col_v.at[j]], add=True)

            @pl.when(t < (NBLK // NBUF) - 1)
            def _():
                pltpu.async_copy(u_hbm.at[row_v.at[j + NBUF]], g_v.at[k],
                                 gsem.at[k])
        return 0
    lax.fori_loop(0, NBLK // NBUF, body, 0)
    plsc.subcore_barrier()
    pltpu.sync_copy(acc_sh.at[pl.ds(s * RPW, RPW)],
                    out_hbm.at[c, pl.ds(s * RPW, RPW)])

    @pl.when(s == 15)
    def _():
        pltpu.sync_copy(acc_sh.at[pl.ds(16 * RPW, 16)],
                        out_hbm.at[c, pl.ds(16 * RPW, 16)])


def kernel(x, x_net, edge_index_node_net, edge_index_net_node, num_instances,
           batch, enc_W1, enc_b1, enc_W2, enc_b2, encn_W1, encn_b1, encn_W2,
           encn_b2, conv_W, conv_b, conv_root, reconv_W, reconv_b, reconv_root,
           ln_gamma, ln_beta):
    f32 = jnp.float32
    row1, col1 = edge_index_node_net[0], edge_index_node_net[1]
    row2, col2 = edge_index_net_node[0], edge_index_net_node[1]
    row1b = row1.reshape(NW, NBLK, EB)
    col1b = col1.reshape(NW, NBLK, EB)
    row2b = row2.reshape(NW, NBLK, EB)
    col2b = col2.reshape(NW, NBLK, EB)

    xp = jnp.pad(x, ((0, 0), (0, 128 - x.shape[1])))
    xnp = jnp.pad(x_net, ((0, 0), (0, 128 - x_net.shape[1])))
    w1t = jnp.pad(enc_W1.T, ((0, 128 - enc_W1.shape[1]), (0, 0)))
    wn1t = jnp.pad(encn_W1.T, ((0, 128 - encn_W1.shape[1]), (0, 0)))
    x_inst = _mlp(xp, w1t, enc_b1[None, :], enc_W2.T, enc_b2[None, :], 1000)
    xn = _mlp(xnp, wn1t, encn_b1[None, :], encn_W2.T, encn_b2[None, :], 1000)
    h0 = jnp.concatenate([x_inst, xn], axis=0)

    partials = _hist_sc(row1, row2)
    dinv_all = _deg_reduce(partials)
    dinv1 = dinv_all[0][:, None]
    dinv2 = dinv_all[1][:, None]

    zeros_slab = jnp.zeros((RPW, EMB), f32)

    h_list = [h0]
    h = h0
    for l in range(NUM_LAYER):
        u1, z1 = _pre(h, conv_W[l].T, conv_b[l][None, :],
                      conv_root[l][None, :], dinv1)
        acc1 = _seg_sc(u1, row1b, col1b, zeros_slab)
        h_mid = _post_ln(acc1, z1, dinv1, ln_gamma[l][None, :],
                         ln_beta[l][None, :])
        u2, z2 = _pre(h_mid, reconv_W[l].T, reconv_b[l][None, :],
                      reconv_root[l][None, :], dinv2)
        acc2 = _seg_sc(u2, row2b, col2b, zeros_slab)
        h = _post_res(acc2, z2, dinv2, h)
        h_list.append(h)

    return jnp.concatenate(h_list, axis=1)


# speedup vs baseline: 24.1734x; 1.0511x over previous
import functools

import jax
import jax.numpy as jnp
from jax import lax
from jax.experimental import pallas as pl
from jax.experimental.pallas import tpu as pltpu
from jax.experimental.pallas import tpu_sc as plsc

N_INST = 8000
N_NET = 2000
N_TOT = N_INST + N_NET
E = 320000
EMB = 128
NUM_LAYER = 2

NW = 32
EPW = E // NW
EB = 100
NBLK = EPW // EB
RPW = 624

_sc_mesh = plsc.VectorSubcoreMesh(core_axis_name="c", subcore_axis_name="s")
_sc_params = pltpu.CompilerParams(needs_layout_passes=False)
_sc_flat_params = pltpu.CompilerParams(needs_layout_passes=False,
                                       use_tc_tiling_on_sc=False)


def _leaky(x):
    return jnp.where(x >= 0, x, 0.1 * x)


BLK = 1000

_full = lambda *dims: pl.BlockSpec(dims, lambda i: (0,) * len(dims))
_rows = lambda *dims: pl.BlockSpec(dims, lambda i: (i,) + (0,) * (len(dims) - 1))


def _dinv(p):
    return lax.rsqrt(jnp.sum(p, axis=1) + 1.0)[:, None]


def _uz(h, wt, b, root, dv, u_ref, z_ref):
    xl = jnp.dot(h, wt, preferred_element_type=jnp.float32) + b
    u_ref[...] = dv * jnp.maximum(xl, 0.0)
    z_ref[...] = (dv * dv) * jnp.maximum(xl + root, 0.0)


def _encpre_body(x_ref, w1a_ref, w1b_ref, b1a_ref, b1b_ref, w2a_ref, w2b_ref,
                 b2a_ref, b2b_ref, wt_ref, b_ref, root_ref, p1_ref,
                 h0_ref, u_ref, z_ref):
    inst = pl.program_id(0) < N_INST // BLK
    w1 = jnp.where(inst, w1a_ref[...], w1b_ref[...])
    b1 = jnp.where(inst, b1a_ref[...], b1b_ref[...])
    w2 = jnp.where(inst, w2a_ref[...], w2b_ref[...])
    b2 = jnp.where(inst, b2a_ref[...], b2b_ref[...])
    h = _leaky(jnp.dot(x_ref[...], w1, preferred_element_type=jnp.float32)
               + b1)
    h0 = _leaky(jnp.dot(h, w2, preferred_element_type=jnp.float32) + b2)
    h0_ref[...] = h0
    _uz(h0, wt_ref[...], b_ref[...], root_ref[...], _dinv(p1_ref[...]),
        u_ref, z_ref)


def _encpre(xin, w1a, w1b, b1a, b1b, w2a, w2b, b2a, b2b, wt, b, root, p1):
    h1 = w1a.shape[1]
    return pl.pallas_call(
        _encpre_body,
        grid=(N_TOT // BLK,),
        in_specs=[
            _rows(BLK, EMB),
            _full(EMB, h1), _full(EMB, h1), _full(1, h1), _full(1, h1),
            _full(h1, EMB), _full(h1, EMB), _full(1, EMB), _full(1, EMB),
            _full(EMB, EMB), _full(1, EMB), _full(1, EMB),
            pl.BlockSpec((BLK, NW), lambda i: (i, 0)),
        ],
        out_specs=[_rows(BLK, EMB)] * 3,
        out_shape=[jax.ShapeDtypeStruct((N_TOT, EMB), jnp.float32)] * 3,
    )(xin, w1a, w1b, b1a, b1b, w2a, w2b, b2a, b2b, wt, b, root, p1)


def _postlnpre_body(acc_ref, z1_ref, p1_ref, p2_ref, g_ref, bt_ref,
                    wt_ref, b_ref, root_ref, u_ref, z_ref):
    cval = _dinv(p1_ref[...]) * (acc_ref[0] + acc_ref[1]) + z1_ref[...]
    mu = jnp.mean(cval, axis=-1, keepdims=True)
    d = cval - mu
    var = jnp.mean(d * d, axis=-1, keepdims=True)
    hm = jnp.maximum(d * lax.rsqrt(var + 1e-5) * g_ref[...] + bt_ref[...],
                     0.0)
    _uz(hm, wt_ref[...], b_ref[...], root_ref[...], _dinv(p2_ref[...]),
        u_ref, z_ref)


def _postlnpre(acc, z1, p1, p2, gamma, beta, wt, b, root):
    return pl.pallas_call(
        _postlnpre_body,
        grid=(N_TOT // BLK,),
        in_specs=[
            pl.BlockSpec((2, BLK, EMB), lambda i: (0, i, 0)),
            _rows(BLK, EMB),
            pl.BlockSpec((BLK, NW), lambda i: (i, 0)),
            pl.BlockSpec((BLK, NW), lambda i: (i, 0)),
            _full(1, EMB), _full(1, EMB),
            _full(EMB, EMB), _full(1, EMB), _full(1, EMB),
        ],
        out_specs=[_rows(BLK, EMB)] * 2,
        out_shape=[jax.ShapeDtypeStruct((N_TOT, EMB), jnp.float32)] * 2,
    )(acc, z1, p1, p2, gamma, beta, wt, b, root)


def _postrespre_body(acc_ref, z2_ref, p2_ref, hprev_ref, wt_ref, b_ref,
                     root_ref, p1_ref, h2_ref, u_ref, z_ref):
    h2 = (_dinv(p2_ref[...]) * (acc_ref[0] + acc_ref[1]) + z2_ref[...]
          + hprev_ref[...])
    h2_ref[...] = h2
    _uz(h2, wt_ref[...], b_ref[...], root_ref[...], _dinv(p1_ref[...]),
        u_ref, z_ref)


def _postrespre(acc, z2, p2, hprev, wt, b, root, p1):
    return pl.pallas_call(
        _postrespre_body,
        grid=(N_TOT // BLK,),
        in_specs=[
            pl.BlockSpec((2, BLK, EMB), lambda i: (0, i, 0)),
            _rows(BLK, EMB),
            pl.BlockSpec((BLK, NW), lambda i: (i, 0)),
            _rows(BLK, EMB),
            _full(EMB, EMB), _full(1, EMB), _full(1, EMB),
            pl.BlockSpec((BLK, NW), lambda i: (i, 0)),
        ],
        out_specs=[_rows(BLK, EMB)] * 3,
        out_shape=[jax.ShapeDtypeStruct((N_TOT, EMB), jnp.float32)] * 3,
    )(acc, z2, p2, hprev, wt, b, root, p1)


def _postres_body(acc_ref, z2_ref, p2_ref, hprev_ref, h2_ref):
    h2_ref[...] = (_dinv(p2_ref[...]) * (acc_ref[0] + acc_ref[1])
                   + z2_ref[...] + hprev_ref[...])


def _postres(acc, z2, p2, hprev):
    return pl.pallas_call(
        _postres_body,
        grid=(N_TOT // BLK,),
        in_specs=[
            pl.BlockSpec((2, BLK, EMB), lambda i: (0, i, 0)),
            _rows(BLK, EMB),
            pl.BlockSpec((BLK, NW), lambda i: (i, 0)),
            _rows(BLK, EMB),
        ],
        out_specs=_rows(BLK, EMB),
        out_shape=jax.ShapeDtypeStruct((N_TOT, EMB), jnp.float32),
    )(acc, z2, p2, hprev)


@functools.partial(
    pl.kernel,
    out_type=jax.ShapeDtypeStruct((2 * NW, N_TOT), jnp.float32),
    mesh=_sc_mesh,
    compiler_params=_sc_params,
    scratch_types=[
        pltpu.VMEM((N_TOT,), jnp.float32),
        pltpu.VMEM((EPW,), jnp.int32),
    ],
)
def _hist_sc(row1_hbm, row2_hbm, out_hbm, hist_v, idx_v):
    c = lax.axis_index("c")
    s = lax.axis_index("s")
    wid = c * 16 + s
    base = wid * EPW
    ones = jnp.ones((16,), jnp.float32)
    zeros = jnp.zeros((16,), jnp.float32)
    for which, row_hbm in enumerate((row1_hbm, row2_hbm)):
        def zbody(i, _):
            hist_v[pl.ds(i * 16, 16)] = zeros
            return 0
        lax.fori_loop(0, N_TOT // 16, zbody, 0)
        pltpu.sync_copy(row_hbm.at[pl.ds(base, EPW)], idx_v)

        def body(i, _):
            idx = idx_v[pl.ds(i * 16, 16)]
            plsc.addupdate_scatter(hist_v, [idx], ones)
            return 0
        lax.fori_loop(0, EPW // 16, body, 0)
        pltpu.sync_copy(hist_v, out_hbm.at[which * NW + wid])


NBUF = 2


@functools.partial(
    pl.kernel,
    out_type=jax.ShapeDtypeStruct((2, N_TOT, EMB), jnp.float32),
    mesh=_sc_mesh,
    compiler_params=_sc_flat_params,
    scratch_types=[
        pltpu.VMEM_SHARED((N_TOT, EMB), jnp.float32),
        pltpu.VMEM((NBLK, EB), jnp.int32),
        pltpu.VMEM((NBLK, EB), jnp.int32),
        pltpu.VMEM((NBUF, EB, EMB), jnp.float32),
        pltpu.SemaphoreType.DMA((NBUF,)),
    ],
)
def _seg_sc(u_hbm, row_hbm, col_hbm, zeros_hbm, out_hbm,
            acc_sh, row_v, col_v, g_v, gsem):
    c = lax.axis_index("c")
    s = lax.axis_index("s")
    pltpu.sync_copy(zeros_hbm.at[pl.ds(0, RPW)], acc_sh.at[pl.ds(s * RPW, RPW)])

    @pl.when(s == 15)
    def _():
        pltpu.sync_copy(zeros_hbm.at[pl.ds(0, 16)],
                        acc_sh.at[pl.ds(16 * RPW, 16)])

    wid = c * 16 + s
    pltpu.sync_copy(row_hbm.at[wid], row_v)
    pltpu.sync_copy(col_hbm.at[wid], col_v)
    plsc.subcore_barrier()

    for k in range(NBUF):
        pltpu.async_copy(u_hbm.at[row_v.at[k]], g_v.at[k], gsem.at[k])

    def body(t, _):
        for k in range(NBUF):
            j = t * NBUF + k
            pltpu.make_async_copy(u_hbm.at[row_v.at[j]], g_v.at[k],
                                  gsem.at[k]).wait()
            pltpu.sync_copy(g_v.at[k], acc_sh.at[col_v.at[j]], add=True)

            @pl.when(t < (NBLK // NBUF) - 1)
            def _():
                pltpu.async_copy(u_hbm.at[row_v.at[j + NBUF]], g_v.at[k],
                                 gsem.at[k])
        return 0
    lax.fori_loop(0, NBLK // NBUF, body, 0)
    plsc.subcore_barrier()
    pltpu.sync_copy(acc_sh.at[pl.ds(s * RPW, RPW)],
                    out_hbm.at[c, pl.ds(s * RPW, RPW)])

    @pl.when(s == 15)
    def _():
        pltpu.sync_copy(acc_sh.at[pl.ds(16 * RPW, 16)],
                        out_hbm.at[c, pl.ds(16 * RPW, 16)])


def kernel(x, x_net, edge_index_node_net, edge_index_net_node, num_instances,
           batch, enc_W1, enc_b1, enc_W2, enc_b2, encn_W1, encn_b1, encn_W2,
           encn_b2, conv_W, conv_b, conv_root, reconv_W, reconv_b, reconv_root,
           ln_gamma, ln_beta):
    f32 = jnp.float32
    row1, col1 = edge_index_node_net[0], edge_index_node_net[1]
    row2, col2 = edge_index_net_node[0], edge_index_net_node[1]
    row1b = row1.reshape(NW, NBLK, EB)
    col1b = col1.reshape(NW, NBLK, EB)
    row2b = row2.reshape(NW, NBLK, EB)
    col2b = col2.reshape(NW, NBLK, EB)

    partials = _hist_sc(row1, row2)
    pt = partials.T
    p1, p2 = pt[:, :NW], pt[:, NW:]

    xin = jnp.concatenate(
        [jnp.pad(x, ((0, 0), (0, EMB - x.shape[1]))),
         jnp.pad(x_net, ((0, 0), (0, EMB - x_net.shape[1])))], axis=0)
    h1w = enc_W1.shape[0]
    w1a = jnp.pad(enc_W1.T, ((0, EMB - enc_W1.shape[1]), (0, 0)))
    w1b = jnp.pad(encn_W1.T, ((0, EMB - encn_W1.shape[1]),
                              (0, h1w - encn_W1.shape[0])))
    b1a = enc_b1[None, :]
    b1b = jnp.pad(encn_b1, (0, h1w - encn_b1.shape[0]))[None, :]
    w2a = enc_W2.T
    w2b = jnp.pad(encn_W2.T, ((0, h1w - encn_W2.shape[1]), (0, 0)))
    b2a = enc_b2[None, :]
    b2b = encn_b2[None, :]

    zeros_slab = jnp.zeros((RPW, EMB), f32)
    wt1 = [conv_W[l].T for l in range(NUM_LAYER)]
    wt2 = [reconv_W[l].T for l in range(NUM_LAYER)]

    h0, u, z = _encpre(xin, w1a, w1b, b1a, b1b, w2a, w2b, b2a, b2b,
                       wt1[0], conv_b[0][None, :], conv_root[0][None, :], p1)
    acc = _seg_sc(u, row1b, col1b, zeros_slab)
    u, z = _postlnpre(acc, z, p1, p2, ln_gamma[0][None, :],
                      ln_beta[0][None, :], wt2[0], reconv_b[0][None, :],
                      reconv_root[0][None, :])
    acc = _seg_sc(u, row2b, col2b, zeros_slab)
    h2a, u, z = _postrespre(acc, z, p2, h0, wt1[1], conv_b[1][None, :],
                            conv_root[1][None, :], p1)
    acc = _seg_sc(u, row1b, col1b, zeros_slab)
    u, z = _postlnpre(acc, z, p1, p2, ln_gamma[1][None, :],
                      ln_beta[1][None, :], wt2[1], reconv_b[1][None, :],
                      reconv_root[1][None, :])
    acc = _seg_sc(u, row2b, col2b, zeros_slab)
    h2b = _postres(acc, z, p2, h2a)

    return jnp.concatenate([h0, h2a, h2b], axis=1)


# speedup vs baseline: 24.7042x; 1.0220x over previous
import functools

import jax
import jax.numpy as jnp
from jax import lax
from jax.experimental import pallas as pl
from jax.experimental.pallas import tpu as pltpu
from jax.experimental.pallas import tpu_sc as plsc

N_INST = 8000
N_NET = 2000
N_TOT = N_INST + N_NET
E = 320000
EMB = 128
NUM_LAYER = 2

NW = 32
EPW = E // NW
EB = 50
NBLK = EPW // EB
RPW = 624

_sc_mesh = plsc.VectorSubcoreMesh(core_axis_name="c", subcore_axis_name="s")
_sc_params = pltpu.CompilerParams(needs_layout_passes=False)
_sc_flat_params = pltpu.CompilerParams(needs_layout_passes=False,
                                       use_tc_tiling_on_sc=False)


def _leaky(x):
    return jnp.where(x >= 0, x, 0.1 * x)


BLK = 1000

_full = lambda *dims: pl.BlockSpec(dims, lambda i: (0,) * len(dims))
_rows = lambda *dims: pl.BlockSpec(dims, lambda i: (i,) + (0,) * (len(dims) - 1))


def _dinv(p):
    return lax.rsqrt(jnp.sum(p, axis=1) + 1.0)[:, None]


def _uz(h, wt, b, root, dv, u_ref, z_ref):
    xl = jnp.dot(h, wt, preferred_element_type=jnp.float32) + b
    u_ref[...] = dv * jnp.maximum(xl, 0.0)
    z_ref[...] = (dv * dv) * jnp.maximum(xl + root, 0.0)


def _encpre_body(x_ref, w1a_ref, w1b_ref, b1a_ref, b1b_ref, w2a_ref, w2b_ref,
                 b2a_ref, b2b_ref, wt_ref, b_ref, root_ref, p1_ref,
                 h0_ref, u_ref, z_ref):
    inst = pl.program_id(0) < N_INST // BLK
    w1 = jnp.where(inst, w1a_ref[...], w1b_ref[...])
    b1 = jnp.where(inst, b1a_ref[...], b1b_ref[...])
    w2 = jnp.where(inst, w2a_ref[...], w2b_ref[...])
    b2 = jnp.where(inst, b2a_ref[...], b2b_ref[...])
    h = _leaky(jnp.dot(x_ref[...], w1, preferred_element_type=jnp.float32)
               + b1)
    h0 = _leaky(jnp.dot(h, w2, preferred_element_type=jnp.float32) + b2)
    h0_ref[...] = h0
    _uz(h0, wt_ref[...], b_ref[...], root_ref[...], _dinv(p1_ref[...]),
        u_ref, z_ref)


def _encpre(xin, w1a, w1b, b1a, b1b, w2a, w2b, b2a, b2b, wt, b, root, p1):
    h1 = w1a.shape[1]
    return pl.pallas_call(
        _encpre_body,
        grid=(N_TOT // BLK,),
        in_specs=[
            _rows(BLK, EMB),
            _full(EMB, h1), _full(EMB, h1), _full(1, h1), _full(1, h1),
            _full(h1, EMB), _full(h1, EMB), _full(1, EMB), _full(1, EMB),
            _full(EMB, EMB), _full(1, EMB), _full(1, EMB),
            pl.BlockSpec((BLK, NW), lambda i: (i, 0)),
        ],
        out_specs=[_rows(BLK, EMB)] * 3,
        out_shape=[jax.ShapeDtypeStruct((N_TOT, EMB), jnp.float32)] * 3,
    )(xin, w1a, w1b, b1a, b1b, w2a, w2b, b2a, b2b, wt, b, root, p1)


def _postlnpre_body(acc_ref, z1_ref, p1_ref, p2_ref, g_ref, bt_ref,
                    wt_ref, b_ref, root_ref, u_ref, z_ref):
    cval = _dinv(p1_ref[...]) * (acc_ref[0] + acc_ref[1]) + z1_ref[...]
    mu = jnp.mean(cval, axis=-1, keepdims=True)
    d = cval - mu
    var = jnp.mean(d * d, axis=-1, keepdims=True)
    hm = jnp.maximum(d * lax.rsqrt(var + 1e-5) * g_ref[...] + bt_ref[...],
                     0.0)
    _uz(hm, wt_ref[...], b_ref[...], root_ref[...], _dinv(p2_ref[...]),
        u_ref, z_ref)


def _postlnpre(acc, z1, p1, p2, gamma, beta, wt, b, root):
    return pl.pallas_call(
        _postlnpre_body,
        grid=(N_TOT // BLK,),
        in_specs=[
            pl.BlockSpec((2, BLK, EMB), lambda i: (0, i, 0)),
            _rows(BLK, EMB),
            pl.BlockSpec((BLK, NW), lambda i: (i, 0)),
            pl.BlockSpec((BLK, NW), lambda i: (i, 0)),
            _full(1, EMB), _full(1, EMB),
            _full(EMB, EMB), _full(1, EMB), _full(1, EMB),
        ],
        out_specs=[_rows(BLK, EMB)] * 2,
        out_shape=[jax.ShapeDtypeStruct((N_TOT, EMB), jnp.float32)] * 2,
    )(acc, z1, p1, p2, gamma, beta, wt, b, root)


def _postrespre_body(acc_ref, z2_ref, p2_ref, hprev_ref, wt_ref, b_ref,
                     root_ref, p1_ref, h2_ref, u_ref, z_ref):
    h2 = (_dinv(p2_ref[...]) * (acc_ref[0] + acc_ref[1]) + z2_ref[...]
          + hprev_ref[...])
    h2_ref[...] = h2
    _uz(h2, wt_ref[...], b_ref[...], root_ref[...], _dinv(p1_ref[...]),
        u_ref, z_ref)


def _postrespre(acc, z2, p2, hprev, wt, b, root, p1):
    return pl.pallas_call(
        _postrespre_body,
        grid=(N_TOT // BLK,),
        in_specs=[
            pl.BlockSpec((2, BLK, EMB), lambda i: (0, i, 0)),
            _rows(BLK, EMB),
            pl.BlockSpec((BLK, NW), lambda i: (i, 0)),
            _rows(BLK, EMB),
            _full(EMB, EMB), _full(1, EMB), _full(1, EMB),
            pl.BlockSpec((BLK, NW), lambda i: (i, 0)),
        ],
        out_specs=[_rows(BLK, EMB)] * 3,
        out_shape=[jax.ShapeDtypeStruct((N_TOT, EMB), jnp.float32)] * 3,
    )(acc, z2, p2, hprev, wt, b, root, p1)


def _postres_body(acc_ref, z2_ref, p2_ref, hprev_ref, h2_ref):
    h2_ref[...] = (_dinv(p2_ref[...]) * (acc_ref[0] + acc_ref[1])
                   + z2_ref[...] + hprev_ref[...])


def _postres(acc, z2, p2, hprev):
    return pl.pallas_call(
        _postres_body,
        grid=(N_TOT // BLK,),
        in_specs=[
            pl.BlockSpec((2, BLK, EMB), lambda i: (0, i, 0)),
            _rows(BLK, EMB),
            pl.BlockSpec((BLK, NW), lambda i: (i, 0)),
            _rows(BLK, EMB),
        ],
        out_specs=_rows(BLK, EMB),
        out_shape=jax.ShapeDtypeStruct((N_TOT, EMB), jnp.float32),
    )(acc, z2, p2, hprev)


@functools.partial(
    pl.kernel,
    out_type=jax.ShapeDtypeStruct((2 * NW, N_TOT), jnp.float32),
    mesh=_sc_mesh,
    compiler_params=_sc_params,
    scratch_types=[
        pltpu.VMEM((N_TOT,), jnp.float32),
        pltpu.VMEM((EPW,), jnp.int32),
    ],
)
def _hist_sc(row1_hbm, row2_hbm, out_hbm, hist_v, idx_v):
    c = lax.axis_index("c")
    s = lax.axis_index("s")
    wid = c * 16 + s
    base = wid * EPW
    ones = jnp.ones((16,), jnp.float32)
    zeros = jnp.zeros((16,), jnp.float32)
    for which, row_hbm in enumerate((row1_hbm, row2_hbm)):
        def zbody(i, _):
            hist_v[pl.ds(i * 16, 16)] = zeros
            return 0
        lax.fori_loop(0, N_TOT // 16, zbody, 0)
        pltpu.sync_copy(row_hbm.at[pl.ds(base, EPW)], idx_v)

        def body(i, _):
            idx = idx_v[pl.ds(i * 16, 16)]
            plsc.addupdate_scatter(hist_v, [idx], ones)
            return 0
        lax.fori_loop(0, EPW // 16, body, 0)
        pltpu.sync_copy(hist_v, out_hbm.at[which * NW + wid])


NBUF = 4


@functools.partial(
    pl.kernel,
    out_type=jax.ShapeDtypeStruct((2, N_TOT, EMB), jnp.float32),
    mesh=_sc_mesh,
    compiler_params=_sc_flat_params,
    scratch_types=[
        pltpu.VMEM_SHARED((N_TOT, EMB), jnp.float32),
        pltpu.VMEM((NBLK, EB), jnp.int32),
        pltpu.VMEM((NBLK, EB), jnp.int32),
        pltpu.VMEM((NBUF, EB, EMB), jnp.float32),
        pltpu.SemaphoreType.DMA((NBUF,)),
        pltpu.SemaphoreType.DMA((NBUF,)),
    ],
)
def _seg_sc(u_hbm, row_hbm, col_hbm, zeros_hbm, out_hbm,
            acc_sh, row_v, col_v, g_v, gsem, ssem):
    c = lax.axis_index("c")
    s = lax.axis_index("s")
    pltpu.sync_copy(zeros_hbm.at[pl.ds(0, RPW)], acc_sh.at[pl.ds(s * RPW, RPW)])

    @pl.when(s == 15)
    def _():
        pltpu.sync_copy(zeros_hbm.at[pl.ds(0, 16)],
                        acc_sh.at[pl.ds(16 * RPW, 16)])

    wid = c * 16 + s
    pltpu.sync_copy(row_hbm.at[wid], row_v)
    pltpu.sync_copy(col_hbm.at[wid], col_v)
    plsc.subcore_barrier()

    def _gather(j, k):
        pltpu.async_copy(u_hbm.at[row_v.at[j]], g_v.at[k], gsem.at[k])

    def _gather_wait(j, k):
        pltpu.make_async_copy(u_hbm.at[row_v.at[j]], g_v.at[k],
                              gsem.at[k]).wait()

    def _scatter(j, k):
        pltpu.async_copy(g_v.at[k], acc_sh.at[col_v.at[j]], ssem.at[k],
                         add=True)

    def _scatter_wait(j, k):
        pltpu.make_async_copy(g_v.at[k], acc_sh.at[col_v.at[j]],
                              ssem.at[k]).wait()

    for k in range(3):
        _gather(k, k)

    def body(t, _):
        for kk in range(NBUF):
            j = t * NBUF + kk
            _gather_wait(j, kk)
            _scatter(j, kk)
            kn = (kk + 3) % NBUF
            if kk == 0:
                @pl.when(t == 0)
                def _():
                    _gather(3, 3)

                @pl.when(jnp.logical_and(t >= 1, j + 3 < NBLK))
                def _():
                    _scatter_wait(j - 1, kn)
                    _gather(j + 3, kn)
            else:
                @pl.when(j + 3 < NBLK)
                def _():
                    _scatter_wait(j - 1, kn)
                    _gather(j + 3, kn)
        return 0
    lax.fori_loop(0, NBLK // NBUF, body, 0)

    for kk in range(NBUF):
        _scatter_wait(NBLK - NBUF + kk, kk)
    plsc.subcore_barrier()
    pltpu.sync_copy(acc_sh.at[pl.ds(s * RPW, RPW)],
                    out_hbm.at[c, pl.ds(s * RPW, RPW)])

    @pl.when(s == 15)
    def _():
        pltpu.sync_copy(acc_sh.at[pl.ds(16 * RPW, 16)],
                        out_hbm.at[c, pl.ds(16 * RPW, 16)])


def kernel(x, x_net, edge_index_node_net, edge_index_net_node, num_instances,
           batch, enc_W1, enc_b1, enc_W2, enc_b2, encn_W1, encn_b1, encn_W2,
           encn_b2, conv_W, conv_b, conv_root, reconv_W, reconv_b, reconv_root,
           ln_gamma, ln_beta):
    f32 = jnp.float32
    row1, col1 = edge_index_node_net[0], edge_index_node_net[1]
    row2, col2 = edge_index_net_node[0], edge_index_net_node[1]
    row1b = row1.reshape(NW, NBLK, EB)
    col1b = col1.reshape(NW, NBLK, EB)
    row2b = row2.reshape(NW, NBLK, EB)
    col2b = col2.reshape(NW, NBLK, EB)

    partials = _hist_sc(row1, row2)
    pt = partials.T
    p1, p2 = pt[:, :NW], pt[:, NW:]

    xin = jnp.concatenate(
        [jnp.pad(x, ((0, 0), (0, EMB - x.shape[1]))),
         jnp.pad(x_net, ((0, 0), (0, EMB - x_net.shape[1])))], axis=0)
    h1w = enc_W1.shape[0]
    w1a = jnp.pad(enc_W1.T, ((0, EMB - enc_W1.shape[1]), (0, 0)))
    w1b = jnp.pad(encn_W1.T, ((0, EMB - encn_W1.shape[1]),
                              (0, h1w - encn_W1.shape[0])))
    b1a = enc_b1[None, :]
    b1b = jnp.pad(encn_b1, (0, h1w - encn_b1.shape[0]))[None, :]
    w2a = enc_W2.T
    w2b = jnp.pad(encn_W2.T, ((0, h1w - encn_W2.shape[1]), (0, 0)))
    b2a = enc_b2[None, :]
    b2b = encn_b2[None, :]

    zeros_slab = jnp.zeros((RPW, EMB), f32)
    wt1 = [conv_W[l].T for l in range(NUM_LAYER)]
    wt2 = [reconv_W[l].T for l in range(NUM_LAYER)]

    h0, u, z = _encpre(xin, w1a, w1b, b1a, b1b, w2a, w2b, b2a, b2b,
                       wt1[0], conv_b[0][None, :], conv_root[0][None, :], p1)
    acc = _seg_sc(u, row1b, col1b, zeros_slab)
    u, z = _postlnpre(acc, z, p1, p2, ln_gamma[0][None, :],
                      ln_beta[0][None, :], wt2[0], reconv_b[0][None, :],
                      reconv_root[0][None, :])
    acc = _seg_sc(u, row2b, col2b, zeros_slab)
    h2a, u, z = _postrespre(acc, z, p2, h0, wt1[1], conv_b[1][None, :],
                            conv_root[1][None, :], p1)
    acc = _seg_sc(u, row1b, col1b, zeros_slab)
    u, z = _postlnpre(acc, z, p1, p2, ln_gamma[1][None, :],
                      ln_beta[1][None, :], wt2[1], reconv_b[1][None, :],
                      reconv_root[1][None, :])
    acc = _seg_sc(u, row2b, col2b, zeros_slab)
    h2b = _postres(acc, z, p2, h2a)

    return jnp.concatenate([h0, h2a, h2b], axis=1)


# speedup vs baseline: 25.2131x; 1.0206x over previous
import functools

import jax
import jax.numpy as jnp
from jax import lax
from jax.experimental import pallas as pl
from jax.experimental.pallas import tpu as pltpu
from jax.experimental.pallas import tpu_sc as plsc

N_INST = 8000
N_NET = 2000
N_TOT = N_INST + N_NET
E = 320000
EMB = 128
NUM_LAYER = 2

NW = 32
EPW = E // NW
EB = 50
NBLK = EPW // EB
RPW = 624

_sc_mesh = plsc.VectorSubcoreMesh(core_axis_name="c", subcore_axis_name="s")
_sc_params = pltpu.CompilerParams(needs_layout_passes=False)
_sc_flat_params = pltpu.CompilerParams(needs_layout_passes=False,
                                       use_tc_tiling_on_sc=False)


def _leaky(x):
    return jnp.where(x >= 0, x, 0.1 * x)


BLK = 1000

_full = lambda *dims: pl.BlockSpec(dims, lambda i: (0,) * len(dims))
_rows = lambda *dims: pl.BlockSpec(dims, lambda i: (i,) + (0,) * (len(dims) - 1))


def _dinv(p):
    return lax.rsqrt(jnp.sum(p, axis=1) + 1.0)[:, None]


def _uz(h, wt, b, root, dv, u_ref, z_ref):
    xl = jnp.dot(h, wt, preferred_element_type=jnp.float32) + b
    u_ref[...] = dv * jnp.maximum(xl, 0.0)
    z_ref[...] = (dv * dv) * jnp.maximum(xl + root, 0.0)


def _encpre_body(x_ref, w1a_ref, w1b_ref, b1a_ref, b1b_ref, w2a_ref, w2b_ref,
                 b2a_ref, b2b_ref, wt_ref, b_ref, root_ref, p1_ref,
                 h0_ref, u_ref, z_ref):
    inst = pl.program_id(0) < N_INST // BLK
    w1 = jnp.where(inst, w1a_ref[...], w1b_ref[...])
    b1 = jnp.where(inst, b1a_ref[...], b1b_ref[...])
    w2 = jnp.where(inst, w2a_ref[...], w2b_ref[...])
    b2 = jnp.where(inst, b2a_ref[...], b2b_ref[...])
    h = _leaky(jnp.dot(x_ref[...], w1, preferred_element_type=jnp.float32)
               + b1)
    h0 = _leaky(jnp.dot(h, w2, preferred_element_type=jnp.float32) + b2)
    h0_ref[...] = h0
    _uz(h0, wt_ref[...], b_ref[...], root_ref[...], _dinv(p1_ref[...]),
        u_ref, z_ref)


def _encpre(xin, w1a, w1b, b1a, b1b, w2a, w2b, b2a, b2b, wt, b, root, p1):
    h1 = w1a.shape[1]
    return pl.pallas_call(
        _encpre_body,
        grid=(N_TOT // BLK,),
        in_specs=[
            _rows(BLK, EMB),
            _full(EMB, h1), _full(EMB, h1), _full(1, h1), _full(1, h1),
            _full(h1, EMB), _full(h1, EMB), _full(1, EMB), _full(1, EMB),
            _full(EMB, EMB), _full(1, EMB), _full(1, EMB),
            pl.BlockSpec((BLK, NW), lambda i: (i, 0)),
        ],
        out_specs=[_rows(BLK, EMB)] * 3,
        out_shape=[jax.ShapeDtypeStruct((N_TOT, EMB), jnp.float32)] * 3,
    )(xin, w1a, w1b, b1a, b1b, w2a, w2b, b2a, b2b, wt, b, root, p1)


def _postlnpre_body(acc_ref, z1_ref, p1_ref, p2_ref, g_ref, bt_ref,
                    wt_ref, b_ref, root_ref, u_ref, z_ref):
    cval = _dinv(p1_ref[...]) * (acc_ref[0] + acc_ref[1]) + z1_ref[...]
    mu = jnp.mean(cval, axis=-1, keepdims=True)
    d = cval - mu
    var = jnp.mean(d * d, axis=-1, keepdims=True)
    hm = jnp.maximum(d * lax.rsqrt(var + 1e-5) * g_ref[...] + bt_ref[...],
                     0.0)
    _uz(hm, wt_ref[...], b_ref[...], root_ref[...], _dinv(p2_ref[...]),
        u_ref, z_ref)


def _postlnpre(acc, z1, p1, p2, gamma, beta, wt, b, root):
    return pl.pallas_call(
        _postlnpre_body,
        grid=(N_TOT // BLK,),
        in_specs=[
            pl.BlockSpec((2, BLK, EMB), lambda i: (0, i, 0)),
            _rows(BLK, EMB),
            pl.BlockSpec((BLK, NW), lambda i: (i, 0)),
            pl.BlockSpec((BLK, NW), lambda i: (i, 0)),
            _full(1, EMB), _full(1, EMB),
            _full(EMB, EMB), _full(1, EMB), _full(1, EMB),
        ],
        out_specs=[_rows(BLK, EMB)] * 2,
        out_shape=[jax.ShapeDtypeStruct((N_TOT, EMB), jnp.float32)] * 2,
    )(acc, z1, p1, p2, gamma, beta, wt, b, root)


def _postrespre_body(acc_ref, z2_ref, p2_ref, hprev_ref, wt_ref, b_ref,
                     root_ref, p1_ref, h2_ref, u_ref, z_ref):
    h2 = (_dinv(p2_ref[...]) * (acc_ref[0] + acc_ref[1]) + z2_ref[...]
          + hprev_ref[...])
    h2_ref[...] = h2
    _uz(h2, wt_ref[...], b_ref[...], root_ref[...], _dinv(p1_ref[...]),
        u_ref, z_ref)


def _postrespre(acc, z2, p2, hprev, wt, b, root, p1):
    return pl.pallas_call(
        _postrespre_body,
        grid=(N_TOT // BLK,),
        in_specs=[
            pl.BlockSpec((2, BLK, EMB), lambda i: (0, i, 0)),
            _rows(BLK, EMB),
            pl.BlockSpec((BLK, NW), lambda i: (i, 0)),
            _rows(BLK, EMB),
            _full(EMB, EMB), _full(1, EMB), _full(1, EMB),
            pl.BlockSpec((BLK, NW), lambda i: (i, 0)),
        ],
        out_specs=[_rows(BLK, EMB)] * 3,
        out_shape=[jax.ShapeDtypeStruct((N_TOT, EMB), jnp.float32)] * 3,
    )(acc, z2, p2, hprev, wt, b, root, p1)


def _postres_body(acc_ref, z2_ref, p2_ref, hprev_ref, h2_ref):
    h2_ref[...] = (_dinv(p2_ref[...]) * (acc_ref[0] + acc_ref[1])
                   + z2_ref[...] + hprev_ref[...])


def _postres(acc, z2, p2, hprev):
    return pl.pallas_call(
        _postres_body,
        grid=(N_TOT // BLK,),
        in_specs=[
            pl.BlockSpec((2, BLK, EMB), lambda i: (0, i, 0)),
            _rows(BLK, EMB),
            pl.BlockSpec((BLK, NW), lambda i: (i, 0)),
            _rows(BLK, EMB),
        ],
        out_specs=_rows(BLK, EMB),
        out_shape=jax.ShapeDtypeStruct((N_TOT, EMB), jnp.float32),
    )(acc, z2, p2, hprev)


@functools.partial(
    pl.kernel,
    out_type=jax.ShapeDtypeStruct((2 * NW, N_TOT), jnp.float32),
    mesh=_sc_mesh,
    compiler_params=_sc_params,
    scratch_types=[
        pltpu.VMEM((N_TOT,), jnp.float32),
        pltpu.VMEM((EPW,), jnp.int32),
    ],
)
def _hist_sc(row1_hbm, row2_hbm, out_hbm, hist_v, idx_v):
    c = lax.axis_index("c")
    s = lax.axis_index("s")
    wid = c * 16 + s
    base = wid * EPW
    ones = jnp.ones((16,), jnp.float32)
    zeros = jnp.zeros((16,), jnp.float32)
    for which, row_hbm in enumerate((row1_hbm, row2_hbm)):
        def zbody(i, _):
            hist_v[pl.ds(i * 16, 16)] = zeros
            return 0
        lax.fori_loop(0, N_TOT // 16, zbody, 0)
        pltpu.sync_copy(row_hbm.at[pl.ds(base, EPW)], idx_v)

        def body(i, _):
            idx = idx_v[pl.ds(i * 16, 16)]
            plsc.addupdate_scatter(hist_v, [idx], ones)
            return 0
        lax.fori_loop(0, EPW // 16, body, 0)
        pltpu.sync_copy(hist_v, out_hbm.at[which * NW + wid])


NBUF = 4


@functools.partial(
    pl.kernel,
    out_type=jax.ShapeDtypeStruct((2, N_TOT, EMB), jnp.float32),
    mesh=_sc_mesh,
    compiler_params=_sc_flat_params,
    scratch_types=[
        pltpu.VMEM_SHARED((N_TOT, EMB), jnp.float32),
        pltpu.VMEM((NBLK, EB), jnp.int32),
        pltpu.VMEM((NBLK, EB), jnp.int32),
        pltpu.VMEM((NBUF, EB, EMB), jnp.float32),
        pltpu.SemaphoreType.DMA((NBUF,)),
        pltpu.SemaphoreType.DMA((NBUF,)),
        pltpu.SemaphoreType.DMA,
        pltpu.SemaphoreType.DMA,
        pltpu.SemaphoreType.DMA,
    ],
)
def _seg_sc(u_hbm, row_hbm, col_hbm, zeros_hbm, out_hbm,
            acc_sh, row_v, col_v, g_v, gsem, ssem, zsem, isem, isem2):
    c = lax.axis_index("c")
    s = lax.axis_index("s")
    pltpu.async_copy(zeros_hbm.at[pl.ds(0, RPW)],
                     acc_sh.at[pl.ds(s * RPW, RPW)], zsem)

    @pl.when(s == 15)
    def _():
        pltpu.async_copy(zeros_hbm.at[pl.ds(0, 16)],
                         acc_sh.at[pl.ds(16 * RPW, 16)], zsem)

    wid = c * 16 + s
    pltpu.async_copy(row_hbm.at[wid], row_v, isem)
    pltpu.async_copy(col_hbm.at[wid], col_v, isem2)
    pltpu.make_async_copy(row_hbm.at[wid], row_v, isem).wait()
    pltpu.make_async_copy(col_hbm.at[wid], col_v, isem2).wait()

    def _gather(j, k):
        pltpu.async_copy(u_hbm.at[row_v.at[j]], g_v.at[k], gsem.at[k])

    def _gather_wait(j, k):
        pltpu.make_async_copy(u_hbm.at[row_v.at[j]], g_v.at[k],
                              gsem.at[k]).wait()

    def _scatter(j, k):
        pltpu.async_copy(g_v.at[k], acc_sh.at[col_v.at[j]], ssem.at[k],
                         add=True)

    def _scatter_wait(j, k):
        pltpu.make_async_copy(g_v.at[k], acc_sh.at[col_v.at[j]],
                              ssem.at[k]).wait()

    for k in range(3):
        _gather(k, k)
    pltpu.make_async_copy(zeros_hbm.at[pl.ds(0, RPW)],
                          acc_sh.at[pl.ds(s * RPW, RPW)], zsem).wait()

    @pl.when(s == 15)
    def _():
        pltpu.make_async_copy(zeros_hbm.at[pl.ds(0, 16)],
                              acc_sh.at[pl.ds(16 * RPW, 16)], zsem).wait()
    plsc.subcore_barrier()

    def body(t, _):
        for kk in range(NBUF):
            j = t * NBUF + kk
            _gather_wait(j, kk)
            _scatter(j, kk)
            kn = (kk + 3) % NBUF
            if kk == 0:
                @pl.when(t == 0)
                def _():
                    _gather(3, 3)

                @pl.when(jnp.logical_and(t >= 1, j + 3 < NBLK))
                def _():
                    _scatter_wait(j - 1, kn)
                    _gather(j + 3, kn)
            else:
                @pl.when(j + 3 < NBLK)
                def _():
                    _scatter_wait(j - 1, kn)
                    _gather(j + 3, kn)
        return 0
    lax.fori_loop(0, NBLK // NBUF, body, 0)

    for kk in range(NBUF):
        _scatter_wait(NBLK - NBUF + kk, kk)
    plsc.subcore_barrier()
    pltpu.sync_copy(acc_sh.at[pl.ds(s * RPW, RPW)],
                    out_hbm.at[c, pl.ds(s * RPW, RPW)])

    @pl.when(s == 15)
    def _():
        pltpu.sync_copy(acc_sh.at[pl.ds(16 * RPW, 16)],
                        out_hbm.at[c, pl.ds(16 * RPW, 16)])


def kernel(x, x_net, edge_index_node_net, edge_index_net_node, num_instances,
           batch, enc_W1, enc_b1, enc_W2, enc_b2, encn_W1, encn_b1, encn_W2,
           encn_b2, conv_W, conv_b, conv_root, reconv_W, reconv_b, reconv_root,
           ln_gamma, ln_beta):
    f32 = jnp.float32
    row1, col1 = edge_index_node_net[0], edge_index_node_net[1]
    row2, col2 = edge_index_net_node[0], edge_index_net_node[1]
    row1b = row1.reshape(NW, NBLK, EB)
    col1b = col1.reshape(NW, NBLK, EB)
    row2b = row2.reshape(NW, NBLK, EB)
    col2b = col2.reshape(NW, NBLK, EB)

    partials = _hist_sc(row1, row2)
    pt = partials.T
    p1, p2 = pt[:, :NW], pt[:, NW:]

    xin = jnp.concatenate(
        [jnp.pad(x, ((0, 0), (0, EMB - x.shape[1]))),
         jnp.pad(x_net, ((0, 0), (0, EMB - x_net.shape[1])))], axis=0)
    h1w = enc_W1.shape[0]
    w1a = jnp.pad(enc_W1.T, ((0, EMB - enc_W1.shape[1]), (0, 0)))
    w1b = jnp.pad(encn_W1.T, ((0, EMB - encn_W1.shape[1]),
                              (0, h1w - encn_W1.shape[0])))
    b1a = enc_b1[None, :]
    b1b = jnp.pad(encn_b1, (0, h1w - encn_b1.shape[0]))[None, :]
    w2a = enc_W2.T
    w2b = jnp.pad(encn_W2.T, ((0, h1w - encn_W2.shape[1]), (0, 0)))
    b2a = enc_b2[None, :]
    b2b = encn_b2[None, :]

    zeros_slab = jnp.zeros((RPW, EMB), f32)
    wt1 = [conv_W[l].T for l in range(NUM_LAYER)]
    wt2 = [reconv_W[l].T for l in range(NUM_LAYER)]

    h0, u, z = _encpre(xin, w1a, w1b, b1a, b1b, w2a, w2b, b2a, b2b,
                       wt1[0], conv_b[0][None, :], conv_root[0][None, :], p1)
    acc = _seg_sc(u, row1b, col1b, zeros_slab)
    u, z = _postlnpre(acc, z, p1, p2, ln_gamma[0][None, :],
                      ln_beta[0][None, :], wt2[0], reconv_b[0][None, :],
                      reconv_root[0][None, :])
    acc = _seg_sc(u, row2b, col2b, zeros_slab)
    h2a, u, z = _postrespre(acc, z, p2, h0, wt1[1], conv_b[1][None, :],
                            conv_root[1][None, :], p1)
    acc = _seg_sc(u, row1b, col1b, zeros_slab)
    u, z = _postlnpre(acc, z, p1, p2, ln_gamma[1][None, :],
                      ln_beta[1][None, :], wt2[1], reconv_b[1][None, :],
                      reconv_root[1][None, :])
    acc = _seg_sc(u, row2b, col2b, zeros_slab)
    h2b = _postres(acc, z, p2, h2a)

    return jnp.concatenate([h0, h2a, h2b], axis=1)
